# baseline (speedup 1.0000x reference)
import dataclasses
import functools

import jax
import jax.numpy as jnp
from jax import lax
from jax.experimental import pallas as pl
from jax.experimental.pallas import tpu as pltpu
from jax.experimental.pallas import tpu_sc as plsc

B, O, N, K = 4096, 3, 20, 2
D, H, L = 128, 4096, 128

JJ = N * K
NC, NS = 2, 16
NW = NC * NS
NV = D // 16

SPLIT_PLAN = ((1024, 8), (1024, 8), (1024, 8), (512, 8), (512, 8))
assert sum(bp for bp, _ in SPLIT_PLAN) == B


@functools.cache
def _sc_pool_fn(segs, ch):
    seg_per_w = segs // NW
    steps = seg_per_w // ch
    rows = ch * JJ
    assert steps % 2 == 0 and seg_per_w % ch == 0

    mesh = plsc.VectorSubcoreMesh(core_axis_name="c", subcore_axis_name="s")
    cp = pltpu.CompilerParams()
    if "needs_layout_passes" in pltpu.CompilerParams.__dataclass_fields__:
        cp = dataclasses.replace(cp, needs_layout_passes=False)

    @functools.partial(
        pl.kernel,
        mesh=mesh,
        compiler_params=cp,
        out_type=jax.ShapeDtypeStruct((segs, D), jnp.float32),
        scratch_types=[
            pltpu.VMEM((rows,), jnp.int32),
            pltpu.VMEM((rows,), jnp.int32),
            pltpu.VMEM((rows,), jnp.float32),
            pltpu.VMEM((rows,), jnp.float32),
            pltpu.VMEM((rows, D), jnp.float32),
            pltpu.VMEM((rows, D), jnp.float32),
            pltpu.VMEM((ch, D), jnp.float32),
            pltpu.VMEM((ch, D), jnp.float32),
            pltpu.SemaphoreType.DMA,
            pltpu.SemaphoreType.DMA,
            pltpu.SemaphoreType.DMA,
            pltpu.SemaphoreType.DMA,
        ],
    )
    def k(emb_hbm, idx_hbm, w_hbm, out_hbm,
          idx0, idx1, w0, w1, rows0, rows1, po0, po1,
          gs0, gs1, os0, os1):
        wid = lax.axis_index("s") * NC + lax.axis_index("c")
        idx_vs, w_vs, rows_vs = (idx0, idx1), (w0, w1), (rows0, rows1)
        po_vs, gsems, osems = (po0, po1), (gs0, gs1), (os0, os1)

        def issue(b, t):
            flat0 = (wid * seg_per_w + t * ch) * JJ
            pltpu.sync_copy(idx_hbm.at[pl.ds(flat0, rows)], idx_vs[b])
            pltpu.sync_copy(w_hbm.at[pl.ds(flat0, rows)], w_vs[b])
            pltpu.async_copy(emb_hbm.at[idx_vs[b]], rows_vs[b], gsems[b])

        def process(b, t, tt):
            pltpu.make_async_copy(
                emb_hbm.at[idx_vs[b]], rows_vs[b], gsems[b]).wait()
            seg0 = wid * seg_per_w + t * ch

            @pl.when(tt >= 1)
            def _():
                pltpu.make_async_copy(
                    po_vs[b], out_hbm.at[pl.ds(seg0, ch)], osems[b]).wait()

            @plsc.parallel_loop(0, ch, unroll=2)
            def _(s):
                base = s * JJ
                acc_a = [None] * NV
                acc_b = [None] * NV
                for j in range(JJ):
                    accs = acc_a if j % 2 == 0 else acc_b
                    wvec = plsc.load_gather(
                        w_vs[b], [jnp.full((16,), base + j, jnp.int32)])
                    for v in range(NV):
                        term = wvec * rows_vs[b][base + j, pl.ds(v * 16, 16)]
                        accs[v] = term if accs[v] is None else accs[v] + term
                for v in range(NV):
                    po_vs[b][s, pl.ds(v * 16, 16)] = acc_a[v] + acc_b[v]

            pltpu.async_copy(po_vs[b], out_hbm.at[pl.ds(seg0, ch)], osems[b])

        issue(0, 0)

        @pl.loop(0, steps // 2)
        def _(tt):
            t = tt * 2
            issue(1, t + 1)
            process(0, t, tt)

            @pl.when(tt < steps // 2 - 1)
            def _():
                issue(0, t + 2)

            process(1, t + 1, tt)

        for b in range(2):
            pltpu.make_async_copy(
                po_vs[b], out_hbm.at[pl.ds(0, ch)], osems[b]).wait()

    return k


def _prep_body(wh_ref, bh_ref, ws_ref, bs_ref, wc_ref, bc_ref):
    wc_ref[...] = 0.5 * lax.dot_general(
        ws_ref[...], wh_ref[...], (((1,), (0,)), ((), ())),
        precision=lax.Precision.HIGHEST,
        preferred_element_type=jnp.float32)
    bc_ref[...] = lax.dot_general(
        bh_ref[...], ws_ref[...], (((1,), (1,)), ((), ())),
        precision=lax.Precision.HIGHEST,
        preferred_element_type=jnp.float32) + bs_ref[...]


def _tc_prepare(W_h, b_h2, W_s, b_s2):
    return pl.pallas_call(
        _prep_body,
        out_shape=(
            jax.ShapeDtypeStruct((L, O * D), jnp.float32),
            jax.ShapeDtypeStruct((1, L), jnp.float32),
        ),
    )(W_h, b_h2, W_s, b_s2)


def _head_body(x_ref, wc_ref, bc_ref, out_ref):
    logits = lax.dot_general(
        x_ref[...], wc_ref[...], (((1,), (1,)), ((), ())),
        precision=lax.Precision.HIGHEST,
        preferred_element_type=jnp.float32) + bc_ref[...]
    m = jnp.max(logits, axis=-1, keepdims=True)
    shifted = logits - m
    out_ref[...] = shifted - jnp.log(
        jnp.sum(jnp.exp(shifted), axis=-1, keepdims=True))


def _tc_head(embed, wc, bc):
    return pl.pallas_call(
        _head_body,
        out_shape=jax.ShapeDtypeStruct((B, L), jnp.float32),
    )(embed, wc, bc)


def kernel(ngrams, weights, emb, W_h, b_h, W_s, b_s):
    xs = []
    b0 = 0
    for bp, ch in SPLIT_PLAN:
        ng_i = jnp.swapaxes(lax.slice_in_dim(ngrams, b0, b0 + bp, axis=0), 2, 3)
        w_i = jnp.swapaxes(lax.slice_in_dim(weights, b0, b0 + bp, axis=0), 2, 3)
        xs.append(_sc_pool_fn(bp * O, ch)(emb, ng_i.reshape(-1), w_i.reshape(-1)))
        b0 += bp
    x = jnp.concatenate(xs, axis=0)
    wc, bc = _tc_prepare(W_h, b_h.reshape(1, H), W_s, b_s.reshape(1, L))
    embed = x.reshape(B, O * D)
    return _tc_head(embed, wc, bc)


# speedup vs baseline: 1.4535x; 1.4535x over previous
import dataclasses
import functools

import jax
import jax.numpy as jnp
from jax import lax
from jax.experimental import pallas as pl
from jax.experimental.pallas import tpu as pltpu
from jax.experimental.pallas import tpu_sc as plsc

B, O, N, K = 4096, 3, 20, 2
D, H, L = 128, 4096, 128

JJ = N * K
NC, NS = 2, 16
NW = NC * NS
NV = D // 16

SPLIT_PLAN = ((1024, 8), (1024, 8), (1024, 8), (512, 8), (512, 8))
assert sum(bp for bp, _ in SPLIT_PLAN) == B


@functools.cache
def _sc_pool_fn(segs, ch):
    seg_per_w = segs // NW
    steps = seg_per_w // ch
    rows = ch * JJ
    assert steps % 2 == 0 and seg_per_w % ch == 0

    mesh = plsc.VectorSubcoreMesh(core_axis_name="c", subcore_axis_name="s")
    cp = pltpu.CompilerParams()
    if "needs_layout_passes" in pltpu.CompilerParams.__dataclass_fields__:
        cp = dataclasses.replace(cp, needs_layout_passes=False)

    @functools.partial(
        pl.kernel,
        mesh=mesh,
        compiler_params=cp,
        out_type=jax.ShapeDtypeStruct((segs, D), jnp.float32),
        scratch_types=[
            pltpu.VMEM((rows,), jnp.int32),
            pltpu.VMEM((rows,), jnp.int32),
            pltpu.VMEM((rows,), jnp.float32),
            pltpu.VMEM((rows,), jnp.float32),
            pltpu.VMEM((rows, D), jnp.float32),
            pltpu.VMEM((rows, D), jnp.float32),
            pltpu.VMEM((ch, D), jnp.float32),
            pltpu.VMEM((ch, D), jnp.float32),
            pltpu.SemaphoreType.DMA,
            pltpu.SemaphoreType.DMA,
            pltpu.SemaphoreType.DMA,
            pltpu.SemaphoreType.DMA,
        ],
    )
    def k(emb_hbm, idx_hbm, w_hbm, out_hbm,
          idx0, idx1, w0, w1, rows0, rows1, po0, po1,
          gs0, gs1, os0, os1):
        wid = lax.axis_index("s") * NC + lax.axis_index("c")
        idx_vs, w_vs, rows_vs = (idx0, idx1), (w0, w1), (rows0, rows1)
        po_vs, gsems, osems = (po0, po1), (gs0, gs1), (os0, os1)

        def issue(b, t):
            flat0 = (wid * seg_per_w + t * ch) * JJ
            pltpu.sync_copy(idx_hbm.at[pl.ds(flat0, rows)], idx_vs[b])
            pltpu.sync_copy(w_hbm.at[pl.ds(flat0, rows)], w_vs[b])
            pltpu.async_copy(emb_hbm.at[idx_vs[b]], rows_vs[b], gsems[b])

        def process(b, t, tt):
            pltpu.make_async_copy(
                emb_hbm.at[idx_vs[b]], rows_vs[b], gsems[b]).wait()
            seg0 = wid * seg_per_w + t * ch

            @pl.when(tt >= 1)
            def _():
                pltpu.make_async_copy(
                    po_vs[b], out_hbm.at[pl.ds(seg0, ch)], osems[b]).wait()

            @pl.loop(0, ch)
            def _(s):
                base = s * JJ
                accs = [None] * NV
                for j in range(JJ):
                    wvec = plsc.load_gather(
                        w_vs[b], [jnp.full((16,), base + j, jnp.int32)])
                    for v in range(NV):
                        term = wvec * rows_vs[b][base + j, pl.ds(v * 16, 16)]
                        accs[v] = term if accs[v] is None else accs[v] + term
                for v in range(NV):
                    po_vs[b][s, pl.ds(v * 16, 16)] = accs[v]

            pltpu.async_copy(po_vs[b], out_hbm.at[pl.ds(seg0, ch)], osems[b])

        issue(0, 0)

        @pl.loop(0, steps // 2)
        def _(tt):
            t = tt * 2
            issue(1, t + 1)
            process(0, t, tt)

            @pl.when(tt < steps // 2 - 1)
            def _():
                issue(0, t + 2)

            process(1, t + 1, tt)

        for b in range(2):
            pltpu.make_async_copy(
                po_vs[b], out_hbm.at[pl.ds(0, ch)], osems[b]).wait()

    return k


def _prep_body(wh_ref, bh_ref, ws_ref, bs_ref, wc_ref, bc_ref):
    wc_ref[...] = 0.5 * lax.dot_general(
        ws_ref[...], wh_ref[...], (((1,), (0,)), ((), ())),
        precision=lax.Precision.HIGHEST,
        preferred_element_type=jnp.float32)
    bc_ref[...] = lax.dot_general(
        bh_ref[...], ws_ref[...], (((1,), (1,)), ((), ())),
        precision=lax.Precision.HIGHEST,
        preferred_element_type=jnp.float32) + bs_ref[...]


def _tc_prepare(W_h, b_h2, W_s, b_s2):
    return pl.pallas_call(
        _prep_body,
        out_shape=(
            jax.ShapeDtypeStruct((L, O * D), jnp.float32),
            jax.ShapeDtypeStruct((1, L), jnp.float32),
        ),
    )(W_h, b_h2, W_s, b_s2)


def _head_body(x_ref, wc_ref, bc_ref, out_ref):
    logits = lax.dot_general(
        x_ref[...], wc_ref[...], (((1,), (1,)), ((), ())),
        precision=lax.Precision.HIGHEST,
        preferred_element_type=jnp.float32) + bc_ref[...]
    m = jnp.max(logits, axis=-1, keepdims=True)
    shifted = logits - m
    out_ref[...] = shifted - jnp.log(
        jnp.sum(jnp.exp(shifted), axis=-1, keepdims=True))


def _tc_head(embed, wc, bc):
    return pl.pallas_call(
        _head_body,
        out_shape=jax.ShapeDtypeStruct((B, L), jnp.float32),
    )(embed, wc, bc)


def kernel(ngrams, weights, emb, W_h, b_h, W_s, b_s):
    xs = []
    b0 = 0
    for bp, ch in SPLIT_PLAN:
        ng_i = jnp.swapaxes(lax.slice_in_dim(ngrams, b0, b0 + bp, axis=0), 2, 3)
        w_i = jnp.swapaxes(lax.slice_in_dim(weights, b0, b0 + bp, axis=0), 2, 3)
        xs.append(_sc_pool_fn(bp * O, ch)(emb, ng_i.reshape(-1), w_i.reshape(-1)))
        b0 += bp
    x = jnp.concatenate(xs, axis=0)
    wc, bc = _tc_prepare(W_h, b_h.reshape(1, H), W_s, b_s.reshape(1, L))
    embed = x.reshape(B, O * D)
    return _tc_head(embed, wc, bc)


# speedup vs baseline: 1.5122x; 1.0404x over previous
import dataclasses
import functools

import jax
import jax.numpy as jnp
from jax import lax
from jax.experimental import pallas as pl
from jax.experimental.pallas import tpu as pltpu
from jax.experimental.pallas import tpu_sc as plsc

B, O, N, K = 4096, 3, 20, 2
D, H, L = 128, 4096, 128

JJ = N * K
NC, NS = 2, 16
NW = NC * NS
NV = D // 16

SPLIT_PLAN = ((512, 8), (1024, 8), (1024, 8), (1024, 8), (512, 8))
assert sum(bp for bp, _ in SPLIT_PLAN) == B


@functools.cache
def _sc_pool_fn(segs, ch):
    seg_per_w = segs // NW
    steps = seg_per_w // ch
    rows = ch * JJ
    assert steps % 2 == 0 and seg_per_w % ch == 0

    mesh = plsc.VectorSubcoreMesh(core_axis_name="c", subcore_axis_name="s")
    cp = pltpu.CompilerParams()
    if "needs_layout_passes" in pltpu.CompilerParams.__dataclass_fields__:
        cp = dataclasses.replace(cp, needs_layout_passes=False)

    @functools.partial(
        pl.kernel,
        mesh=mesh,
        compiler_params=cp,
        out_type=jax.ShapeDtypeStruct((segs, D), jnp.float32),
        scratch_types=[
            pltpu.VMEM((rows,), jnp.int32),
            pltpu.VMEM((rows,), jnp.int32),
            pltpu.VMEM((rows,), jnp.float32),
            pltpu.VMEM((rows,), jnp.float32),
            pltpu.VMEM((rows, D), jnp.float32),
            pltpu.VMEM((rows, D), jnp.float32),
            pltpu.VMEM((ch, D), jnp.float32),
            pltpu.VMEM((ch, D), jnp.float32),
            pltpu.SemaphoreType.DMA,
            pltpu.SemaphoreType.DMA,
            pltpu.SemaphoreType.DMA,
            pltpu.SemaphoreType.DMA,
        ],
    )
    def k(emb_hbm, idx_hbm, w_hbm, out_hbm,
          idx0, idx1, w0, w1, rows0, rows1, po0, po1,
          gs0, gs1, os0, os1):
        wid = lax.axis_index("s") * NC + lax.axis_index("c")
        idx_vs, w_vs, rows_vs = (idx0, idx1), (w0, w1), (rows0, rows1)
        po_vs, gsems, osems = (po0, po1), (gs0, gs1), (os0, os1)

        def issue(b, t):
            flat0 = (wid * seg_per_w + t * ch) * JJ
            pltpu.sync_copy(idx_hbm.at[pl.ds(flat0, rows)], idx_vs[b])
            pltpu.sync_copy(w_hbm.at[pl.ds(flat0, rows)], w_vs[b])
            pltpu.async_copy(emb_hbm.at[idx_vs[b]], rows_vs[b], gsems[b])

        def process(b, t, tt):
            pltpu.make_async_copy(
                emb_hbm.at[idx_vs[b]], rows_vs[b], gsems[b]).wait()
            seg0 = wid * seg_per_w + t * ch

            @pl.when(tt >= 1)
            def _():
                pltpu.make_async_copy(
                    po_vs[b], out_hbm.at[pl.ds(seg0, ch)], osems[b]).wait()

            @pl.loop(0, ch)
            def _(s):
                base = s * JJ
                accs = [None] * NV
                for j in range(JJ):
                    wvec = plsc.load_gather(
                        w_vs[b], [jnp.full((16,), base + j, jnp.int32)])
                    for v in range(NV):
                        term = wvec * rows_vs[b][base + j, pl.ds(v * 16, 16)]
                        accs[v] = term if accs[v] is None else accs[v] + term
                for v in range(NV):
                    po_vs[b][s, pl.ds(v * 16, 16)] = accs[v]

            pltpu.async_copy(po_vs[b], out_hbm.at[pl.ds(seg0, ch)], osems[b])

        issue(0, 0)

        @pl.loop(0, steps // 2)
        def _(tt):
            t = tt * 2
            issue(1, t + 1)
            process(0, t, tt)

            @pl.when(tt < steps // 2 - 1)
            def _():
                issue(0, t + 2)

            process(1, t + 1, tt)

        for b in range(2):
            pltpu.make_async_copy(
                po_vs[b], out_hbm.at[pl.ds(0, ch)], osems[b]).wait()

    return k


def _prep_body(wh_ref, bh_ref, ws_ref, bs_ref, wc_ref, bc_ref):
    wc_ref[...] = 0.5 * lax.dot_general(
        ws_ref[...], wh_ref[...], (((1,), (0,)), ((), ())),
        precision=lax.Precision.HIGHEST,
        preferred_element_type=jnp.float32)
    bc_ref[...] = lax.dot_general(
        bh_ref[...], ws_ref[...], (((1,), (1,)), ((), ())),
        precision=lax.Precision.HIGHEST,
        preferred_element_type=jnp.float32) + bs_ref[...]


def _tc_prepare(W_h, b_h2, W_s, b_s2):
    return pl.pallas_call(
        _prep_body,
        out_shape=(
            jax.ShapeDtypeStruct((L, O * D), jnp.float32),
            jax.ShapeDtypeStruct((1, L), jnp.float32),
        ),
    )(W_h, b_h2, W_s, b_s2)


def _head_body(x_ref, wc_ref, bc_ref, out_ref):
    logits = lax.dot_general(
        x_ref[...], wc_ref[...], (((1,), (1,)), ((), ())),
        precision=lax.Precision.HIGHEST,
        preferred_element_type=jnp.float32) + bc_ref[...]
    m = jnp.max(logits, axis=-1, keepdims=True)
    shifted = logits - m
    out_ref[...] = shifted - jnp.log(
        jnp.sum(jnp.exp(shifted), axis=-1, keepdims=True))


def _tc_head(embed, wc, bc, rows):
    return pl.pallas_call(
        _head_body,
        out_shape=jax.ShapeDtypeStruct((rows, L), jnp.float32),
    )(embed, wc, bc)


def kernel(ngrams, weights, emb, W_h, b_h, W_s, b_s):
    wc, bc = _tc_prepare(W_h, b_h.reshape(1, H), W_s, b_s.reshape(1, L))
    outs = []
    b0 = 0
    for bp, ch in SPLIT_PLAN:
        ng_i = jnp.swapaxes(lax.slice_in_dim(ngrams, b0, b0 + bp, axis=0), 2, 3)
        w_i = jnp.swapaxes(lax.slice_in_dim(weights, b0, b0 + bp, axis=0), 2, 3)
        x_i = _sc_pool_fn(bp * O, ch)(emb, ng_i.reshape(-1), w_i.reshape(-1))
        outs.append(_tc_head(x_i.reshape(bp, O * D), wc, bc, bp))
        b0 += bp
    return jnp.concatenate(outs, axis=0)
